# baseline (speedup 1.0000x reference)
import functools

import jax
import jax.numpy as jnp
from jax import lax
from jax.experimental import pallas as pl
from jax.experimental.pallas import tpu as pltpu
from jax.experimental.pallas import tpu_sc as plsc

N_E = 8192
DIM = 256
BETA = 0.25
T = 8192

BM = 512
BN = 1024
KT = N_E // BN
TT = T // BM

_NC = 2
_NS = 16
_NW = _NC * _NS
_CH = 128


CW = 1024
NCH = N_E // CW
NG = CW // 128


def _dist_argmin_body(z_ref, embT_ref, emb2_ref, idx_ref, zf_ref):
    zb4 = z_ref[0]
    zf = jnp.transpose(zb4.reshape(DIM, BM), (1, 0))
    zf_ref[...] = zf
    zf2 = jnp.sum(zf * zf, axis=1, keepdims=True)
    lane = lax.broadcasted_iota(jnp.int32, (BM, 128), 1)
    acc_v = None
    acc_g = None
    for c in range(NCH):
        dotc = lax.dot_general(zf, embT_ref[:, pl.ds(c * CW, CW)],
                               (((1,), (0,)), ((), ())),
                               preferred_element_type=jnp.float32)
        for g in range(NG):
            gid = c * NG + g
            d2 = ((zf2 - dotc[:, g * 128:(g + 1) * 128])
                  + emb2_ref[:, pl.ds(gid * 128, 128)])
            x = jnp.maximum(d2, 0.0)
            dist = x * lax.rsqrt(x)
            if gid == 0:
                acc_v = dist
                acc_g = jnp.zeros((BM, 128), jnp.int32)
            else:
                t = dist < acc_v
                acc_v = jnp.where(t, dist, acc_v)
                acc_g = jnp.where(t, jnp.full((BM, 128), gid, jnp.int32),
                                  acc_g)
    m = jnp.min(acc_v, axis=1, keepdims=True)
    gcol = acc_g * 128 + lane
    cand = jnp.where(acc_v == m, gcol, jnp.int32(1 << 30))
    idx_ref[...] = jnp.min(cand, axis=1, keepdims=True)


def _dist_argmin(z, embT, emb2):
    return pl.pallas_call(
        _dist_argmin_body,
        grid=(TT,),
        in_specs=[
            pl.BlockSpec((1, DIM, 16, 32), lambda t: (t // 2, 0, t % 2, 0)),
            pl.BlockSpec((DIM, N_E), lambda t: (0, 0)),
            pl.BlockSpec((1, N_E), lambda t: (0, 0)),
        ],
        out_specs=[
            pl.BlockSpec((BM, 1), lambda t: (t, 0)),
            pl.BlockSpec((BM, DIM), lambda t: (t, 0)),
        ],
        out_shape=[
            jax.ShapeDtypeStruct((T, 1), jnp.int32),
            jax.ShapeDtypeStruct((T, DIM), jnp.float32),
        ],
    )(z, embT, emb2)


def _gather_sc(emb, idx):
    b_per_w = T // _NW
    n_ch = b_per_w // _CH
    mesh = plsc.VectorSubcoreMesh(core_axis_name="c", subcore_axis_name="s")

    @functools.partial(
        pl.kernel,
        out_type=jax.ShapeDtypeStruct((T, DIM), jnp.float32),
        mesh=mesh,
        scratch_types=[
            pltpu.VMEM((_CH,), jnp.int32),
            pltpu.VMEM((_CH, DIM), jnp.float32),
            pltpu.SemaphoreType.DMA,
        ],
    )
    def k(emb_hbm, idx_hbm, out_hbm, idx_v, rows_v, sem):
        wid = lax.axis_index("s") * _NC + lax.axis_index("c")
        base = wid * b_per_w
        for c in range(n_ch):
            off = base + c * _CH
            pltpu.sync_copy(idx_hbm.at[pl.ds(off, _CH)], idx_v)
            pltpu.async_copy(emb_hbm.at[idx_v], rows_v, sem).wait()
            pltpu.sync_copy(rows_v, out_hbm.at[pl.ds(off, _CH)])

    return k(emb, idx)


def _st_loss_body(zf_ref, zq_ref, st_ref, loss_ref, acc_ref):
    t = pl.program_id(0)
    zf = zf_ref[...]
    d = zq_ref[...] - zf
    st_ref[...] = zf + d
    s = jnp.sum(d * d)

    @pl.when(t == 0)
    def _():
        acc_ref[0, 0] = 0.0

    acc_ref[0, 0] = acc_ref[0, 0] + s

    @pl.when(t == TT - 1)
    def _():
        m = acc_ref[0, 0] * (1.0 / (T * DIM))
        loss_ref[0, 0] = BETA * m + m


def _st_loss(zf, zq):
    return pl.pallas_call(
        _st_loss_body,
        grid=(TT,),
        in_specs=[
            pl.BlockSpec((BM, DIM), lambda t: (t, 0)),
            pl.BlockSpec((BM, DIM), lambda t: (t, 0)),
        ],
        out_specs=[
            pl.BlockSpec((BM, DIM), lambda t: (t, 0)),
            pl.BlockSpec(memory_space=pltpu.SMEM),
        ],
        out_shape=[
            jax.ShapeDtypeStruct((T, DIM), jnp.float32),
            jax.ShapeDtypeStruct((1, 1), jnp.float32),
        ],
        scratch_shapes=[pltpu.SMEM((1, 1), jnp.float32)],
    )(zf, zq)


def kernel(z, emb):
    emb2 = jnp.sum(emb * emb, axis=1)[None, :]
    embT2 = emb.T * 2.0
    idx2, zf = _dist_argmin(z, embT2, emb2)
    idx = idx2.reshape(T)
    zq = _gather_sc(emb, idx)
    st, loss = _st_loss(zf, zq)
    z_q_out = jnp.transpose(st.reshape(8, 32, 32, DIM), (0, 3, 1, 2))
    return (z_q_out, loss.reshape(()))


# speedup vs baseline: 1.3694x; 1.3694x over previous
import functools

import jax
import jax.numpy as jnp
from jax import lax
from jax.experimental import pallas as pl
from jax.experimental.pallas import tpu as pltpu
from jax.experimental.pallas import tpu_sc as plsc

N_E = 8192
DIM = 256
BETA = 0.25
T = 8192

BM = 1024
BN = 1024
KT = N_E // BN
TT = T // BM

_NC = 2
_NS = 16
_NW = _NC * _NS
_CH = 128


CW = 1024
NCH = N_E // CW
NG = CW // 128


def _dist_argmin_body(zf_ref, zf2_ref, embT_ref, emb2_ref, idx_ref):
    zf = zf_ref[...]
    zf2 = zf2_ref[...]
    lane = lax.broadcasted_iota(jnp.int32, (BM, 128), 1)
    acc_v = None
    acc_g = None
    for c in range(NCH):
        dotc = lax.dot_general(zf, embT_ref[:, pl.ds(c * CW, CW)],
                               (((1,), (0,)), ((), ())),
                               preferred_element_type=jnp.float32)
        for g in range(NG):
            gid = c * NG + g
            x = ((zf2 - dotc[:, g * 128:(g + 1) * 128])
                 + emb2_ref[:, pl.ds(gid * 128, 128)])
            dist = x * lax.rsqrt(x)
            if gid == 0:
                acc_v = dist
                acc_g = jnp.zeros((BM, 128), jnp.int32)
            else:
                t = dist < acc_v
                acc_v = jnp.where(t, dist, acc_v)
                acc_g = jnp.where(t, jnp.full((BM, 128), gid, jnp.int32),
                                  acc_g)
    m = jnp.min(acc_v, axis=1, keepdims=True)
    gcol = acc_g * 128 + lane
    cand = jnp.where(acc_v == m, gcol, jnp.int32(1 << 30))
    idx_ref[...] = jnp.min(cand, axis=1, keepdims=True)


def _dist_argmin(zf, zf2, embT, emb2):
    return pl.pallas_call(
        _dist_argmin_body,
        grid=(TT,),
        in_specs=[
            pl.BlockSpec((BM, DIM), lambda t: (t, 0)),
            pl.BlockSpec((BM, 1), lambda t: (t, 0)),
            pl.BlockSpec((DIM, N_E), lambda t: (0, 0)),
            pl.BlockSpec((1, N_E), lambda t: (0, 0)),
        ],
        out_specs=pl.BlockSpec((BM, 1), lambda t: (t, 0)),
        out_shape=jax.ShapeDtypeStruct((T, 1), jnp.int32),
    )(zf, zf2, embT, emb2)


def _gather_sc(emb, idx):
    b_per_w = T // _NW
    n_ch = b_per_w // _CH
    mesh = plsc.VectorSubcoreMesh(core_axis_name="c", subcore_axis_name="s")

    @functools.partial(
        pl.kernel,
        out_type=jax.ShapeDtypeStruct((T, DIM), jnp.float32),
        mesh=mesh,
        scratch_types=[
            pltpu.VMEM((_CH,), jnp.int32),
            pltpu.VMEM((_CH, DIM), jnp.float32),
            pltpu.SemaphoreType.DMA,
        ],
    )
    def k(emb_hbm, idx_hbm, out_hbm, idx_v, rows_v, sem):
        wid = lax.axis_index("s") * _NC + lax.axis_index("c")
        base = wid * b_per_w
        for c in range(n_ch):
            off = base + c * _CH
            pltpu.sync_copy(idx_hbm.at[pl.ds(off, _CH)], idx_v)
            pltpu.async_copy(emb_hbm.at[idx_v], rows_v, sem).wait()
            pltpu.sync_copy(rows_v, out_hbm.at[pl.ds(off, _CH)])

    return k(emb, idx)


def _st_loss_body(zf_ref, zq_ref, st_ref, loss_ref, acc_ref):
    t = pl.program_id(0)
    zf = zf_ref[...]
    d = zq_ref[...] - zf
    st_ref[...] = zf + d
    s = jnp.sum(d * d)

    @pl.when(t == 0)
    def _():
        acc_ref[0, 0] = 0.0

    acc_ref[0, 0] = acc_ref[0, 0] + s

    @pl.when(t == TT - 1)
    def _():
        m = acc_ref[0, 0] * (1.0 / (T * DIM))
        loss_ref[0, 0] = BETA * m + m


def _st_loss(zf, zq):
    return pl.pallas_call(
        _st_loss_body,
        grid=(TT,),
        in_specs=[
            pl.BlockSpec((BM, DIM), lambda t: (t, 0)),
            pl.BlockSpec((BM, DIM), lambda t: (t, 0)),
        ],
        out_specs=[
            pl.BlockSpec((BM, DIM), lambda t: (t, 0)),
            pl.BlockSpec(memory_space=pltpu.SMEM),
        ],
        out_shape=[
            jax.ShapeDtypeStruct((T, DIM), jnp.float32),
            jax.ShapeDtypeStruct((1, 1), jnp.float32),
        ],
        scratch_shapes=[pltpu.SMEM((1, 1), jnp.float32)],
    )(zf, zq)


def kernel(z, emb):
    zp = jnp.transpose(z, (0, 2, 3, 1))
    zf = zp.reshape(-1, DIM)
    zf2 = jnp.sum(zf * zf, axis=1, keepdims=True)
    emb2 = jnp.sum(emb * emb, axis=1)[None, :]
    embT2 = emb.T * 2.0
    idx2 = _dist_argmin(zf, zf2, embT2, emb2)
    idx = idx2.reshape(T)
    zq = _gather_sc(emb, idx)
    st, loss = _st_loss(zf, zq)
    z_q_out = jnp.transpose(st.reshape(zp.shape), (0, 3, 1, 2))
    return (z_q_out, loss.reshape(()))
